# baseline (speedup 1.0000x reference)
import functools

import jax
import jax.numpy as jnp
from jax import lax
from jax.experimental import pallas as pl
from jax.experimental.pallas import tpu as pltpu
from jax.experimental.pallas import tpu_sc as plsc

VOCAB = 1000000
D = 64
B = 4096
S = 200
PADW = 128
NC, NS = 2, 16
NW = NC * NS
BPW = B // NW
POS_FLAT = S * D
LANES = 16
JB = BPW // LANES
NSTREAM = 16
RPS = BPW // NSTREAM
NBUF = 2

_mesh = plsc.VectorSubcoreMesh(core_axis_name="c", subcore_axis_name="s")


@functools.partial(
    pl.kernel,
    mesh=_mesh,
    out_type=jax.ShapeDtypeStruct((S, D, B), jnp.float32),
    scratch_types=[
        pltpu.VMEM((S, BPW), jnp.int32),
        [pltpu.VMEM((BPW, PADW), jnp.float32)] * NBUF,
        [pltpu.VMEM((D, BPW), jnp.float32)] * NBUF,
        pltpu.VMEM((POS_FLAT,), jnp.float32),
        [pltpu.SemaphoreType.DMA] * NBUF,
        [pltpu.SemaphoreType.DMA] * NBUF,
    ],
    compiler_params=pltpu.CompilerParams(
        use_tc_tiling_on_sc=True, needs_layout_passes=False),
)
def _embed_sc(ids_hbm, tab_hbm, pos_hbm, out_hbm,
              idt, gbufs, tbufs, posv, gsems, psems):
    wid = lax.axis_index("s") * NC + lax.axis_index("c")
    b0 = wid * BPW

    pltpu.sync_copy(ids_hbm.at[:, pl.ds(b0, BPW)], idt)
    pltpu.sync_copy(pos_hbm.at[pl.ds(0, POS_FLAT)], posv)

    lanes = lax.iota(jnp.int32, LANES)
    rows_jb = [jb * LANES + lanes for jb in range(JB)]

    def fire(s, k):
        for st in range(NSTREAM):
            pltpu.async_copy(
                tab_hbm.at[idt.at[s, pl.ds(st * RPS, RPS)]],
                gbufs[k].at[pl.ds(st * RPS, RPS)],
                gsems[k],
            )

    def drain(k):
        pltpu.make_async_copy(
            tab_hbm.at[pl.ds(0, BPW)], gbufs[k], gsems[k]).wait()

    cvecs = [c * LANES + lanes for c in range(D // LANES)]

    def process(s, k):
        gbuf, tbuf = gbufs[k], tbufs[k]
        posc = [posv[pl.ds(s * D + c * LANES, LANES)] for c in range(D // LANES)]

        @plsc.parallel_loop(0, BPW, 1, unroll=4)
        def _(j):
            js = jnp.broadcast_to(j, (LANES,))
            for c in range(D // LANES):
                val = gbuf[j, pl.ds(c * LANES, LANES)] + posc[c]
                plsc.store_scatter(tbuf, [cvecs[c], js], val)

    def put(s, k):
        pltpu.async_copy(tbufs[k], out_hbm.at[s, :, pl.ds(b0, BPW)], psems[k])

    def wait_put(k):
        pltpu.make_async_copy(
            tbufs[k], out_hbm.at[0, :, pl.ds(b0, BPW)], psems[k]).wait()

    for s in range(NBUF):
        fire(s, s)

    def ring(i, c):
        sb = NBUF * i
        for k in range(NBUF):
            s = sb + k
            drain(k)

            @pl.when(s >= NBUF)
            def _():
                wait_put(k)

            process(s, k)
            put(s, k)

            @pl.when(s + NBUF < S)
            def _():
                fire(s + NBUF, k)
        return c

    niter = (S - 2) // NBUF
    lax.fori_loop(0, niter, ring, 0)
    for s in range(niter * NBUF, S):
        k = s % NBUF
        drain(k)
        wait_put(k)
        process(s, k)
        put(s, k)
    for k in range(NBUF):
        wait_put(k)


def kernel(input_ids, token_table, pos_table):
    idsT = input_ids.T.astype(jnp.int32)
    tpad = jnp.pad(token_table, ((0, 0), (0, PADW - D)))
    pos1 = pos_table.reshape(-1)
    out = _embed_sc(idsT, tpad, pos1)
    return out.transpose(2, 0, 1)


# speedup vs baseline: 1.0290x; 1.0290x over previous
import functools

import jax
import jax.numpy as jnp
from jax import lax
from jax.experimental import pallas as pl
from jax.experimental.pallas import tpu as pltpu
from jax.experimental.pallas import tpu_sc as plsc

VOCAB = 1000000
D = 64
B = 4096
S = 200
PADW = 128
NC, NS = 2, 16
NW = NC * NS
BPW = B // NW
POS_FLAT = S * D
LANES = 16
JB = BPW // LANES
NSTREAM = 16
RPS = BPW // NSTREAM
NBUF = 3

_mesh = plsc.VectorSubcoreMesh(core_axis_name="c", subcore_axis_name="s")


@functools.partial(
    pl.kernel,
    mesh=_mesh,
    out_type=jax.ShapeDtypeStruct((S, D, B), jnp.float32),
    scratch_types=[
        pltpu.VMEM((S, BPW), jnp.int32),
        [pltpu.VMEM((BPW, PADW), jnp.float32)] * NBUF,
        [pltpu.VMEM((D, BPW), jnp.float32)] * NBUF,
        pltpu.VMEM((POS_FLAT,), jnp.float32),
        [pltpu.SemaphoreType.DMA] * NBUF,
        [pltpu.SemaphoreType.DMA] * NBUF,
    ],
    compiler_params=pltpu.CompilerParams(
        use_tc_tiling_on_sc=True, needs_layout_passes=False),
)
def _embed_sc(ids_hbm, tab_hbm, pos_hbm, out_hbm,
              idt, gbufs, tbufs, posv, gsems, psems):
    wid = lax.axis_index("s") * NC + lax.axis_index("c")
    b0 = wid * BPW

    pltpu.sync_copy(ids_hbm.at[:, pl.ds(b0, BPW)], idt)
    pltpu.sync_copy(pos_hbm.at[pl.ds(0, POS_FLAT)], posv)

    lanes = lax.iota(jnp.int32, LANES)
    rows_jb = [jb * LANES + lanes for jb in range(JB)]

    def fire(s, k):
        for st in range(NSTREAM):
            pltpu.async_copy(
                tab_hbm.at[idt.at[s, pl.ds(st * RPS, RPS)]],
                gbufs[k].at[pl.ds(st * RPS, RPS)],
                gsems[k],
            )

    def drain(k):
        pltpu.make_async_copy(
            tab_hbm.at[pl.ds(0, BPW)], gbufs[k], gsems[k]).wait()

    def process(s, k):
        gbuf, tbuf = gbufs[k], tbufs[k]

        @plsc.parallel_loop(0, D, 1, unroll=8)
        def _(d):
            psplat = plsc.load_gather(
                posv, [jnp.broadcast_to(s * D + d, (LANES,))])
            cols = jnp.broadcast_to(d, (LANES,))
            for jb in range(JB):
                val = plsc.load_gather(gbuf, [rows_jb[jb], cols])
                tbuf[d, pl.ds(jb * LANES, LANES)] = val + psplat

    def put(s, k):
        pltpu.async_copy(tbufs[k], out_hbm.at[s, :, pl.ds(b0, BPW)], psems[k])

    def wait_put(k):
        pltpu.make_async_copy(
            tbufs[k], out_hbm.at[0, :, pl.ds(b0, BPW)], psems[k]).wait()

    for s in range(NBUF):
        fire(s, s)

    def ring(i, c):
        sb = NBUF * i
        for k in range(NBUF):
            s = sb + k
            drain(k)

            @pl.when(s >= NBUF)
            def _():
                wait_put(k)

            process(s, k)
            put(s, k)

            @pl.when(s + NBUF < S)
            def _():
                fire(s + NBUF, k)
        return c

    niter = (S - 2) // NBUF
    lax.fori_loop(0, niter, ring, 0)
    for s in range(niter * NBUF, S):
        k = s % NBUF
        drain(k)
        wait_put(k)
        process(s, k)
        put(s, k)
    for k in range(NBUF):
        wait_put(k)


def kernel(input_ids, token_table, pos_table):
    idsT = input_ids.T.astype(jnp.int32)
    tpad = jnp.pad(token_table, ((0, 0), (0, PADW - D)))
    pos1 = pos_table.reshape(-1)
    out = _embed_sc(idsT, tpad, pos1)
    return out.transpose(2, 0, 1)
